# baseline (speedup 1.0000x reference)
import functools

import jax
import jax.numpy as jnp
from jax import lax
from jax.experimental import pallas as pl
from jax.experimental.pallas import tpu as pltpu
from jax.experimental.pallas import tpu_sc as plsc

T = 2048
D = 768
E = 16
H = 2048
BT = 256
NB = T // BT + E
PT = NB * BT
NC = 2
NS = 16
NW = NC * NS
SLOTS = PT // NW
TPW = T // NW


def _routing_body(x_ref, wg_ref, rank_ref, be_ref):
    x = x_ref[...]
    wg = wg_ref[...]
    logits = jnp.dot(x, wg, preferred_element_type=jnp.float32)
    mx = jnp.max(logits, axis=1, keepdims=True)
    iota_e = lax.broadcasted_iota(jnp.int32, (T, E), 1)
    eid = jnp.min(jnp.where(logits == mx, iota_e, E), axis=1, keepdims=True)
    onehot = (iota_e == eid).astype(jnp.float32)

    counts = jnp.sum(onehot, axis=0, keepdims=True)
    ci = counts.astype(jnp.int32)
    pc = (((ci + BT - 1) // BT) * BT).astype(jnp.float32)
    ir = lax.broadcasted_iota(jnp.int32, (E, E), 0)
    ic = lax.broadcasted_iota(jnp.int32, (E, E), 1)
    excl = (ir < ic).astype(jnp.float32)
    pstarts = jnp.dot(pc, excl, preferred_element_type=jnp.float32)
    pends = pstarts + pc

    sel_start = jnp.sum(onehot * pstarts, axis=1)

    ir2 = lax.broadcasted_iota(jnp.int32, (BT, BT), 0)
    ic2 = lax.broadcasted_iota(jnp.int32, (BT, BT), 1)
    ltri = (ic2 < ir2).astype(jnp.float32)
    base = jnp.zeros((1, E), jnp.float32)
    for k in range(T // BT):
        blk = onehot[k * BT:(k + 1) * BT, :]
        cum = jnp.dot(ltri, blk, preferred_element_type=jnp.float32) + base
        r = jnp.sum(cum * blk, axis=1) + sel_start[k * BT:(k + 1) * BT]
        rank_ref[k, :] = r.astype(jnp.int32)
        base = base + jnp.sum(blk, axis=0, keepdims=True)

    total = pends[0:1, E - 1:E]
    bb_raw = lax.broadcasted_iota(jnp.int32, (1, 128), 1).astype(jnp.float32) * BT
    bb = jnp.minimum(bb_raw, total - 1.0)
    acc = jnp.zeros((1, 128), jnp.int32)
    for e in range(E):
        acc = acc + (bb >= pends[0:1, e:e + 1]).astype(jnp.int32)
    be_ref[0:1, :] = jnp.clip(acc, 0, E - 1)
    be_ref[1:2, :] = (bb_raw < total).astype(jnp.int32)
    nb_act = total * (1.0 / BT)
    biota = lax.broadcasted_iota(jnp.int32, (1, 128), 1).astype(jnp.float32)
    be_ref[2:3, :] = jnp.minimum(biota, nb_act - 1.0).astype(jnp.int32)


_routing = pl.pallas_call(
    _routing_body,
    out_shape=(
        jax.ShapeDtypeStruct((T // BT, BT), jnp.int32),
        jax.ShapeDtypeStruct((3, 128), jnp.int32),
    ),
)


def _dispatch_body(rank_hbm, x_hbm, xg_hbm, rk_v, rows_v, sem):
    wid = lax.axis_index("s") * NC + lax.axis_index("c")
    base = wid * TPW
    pltpu.sync_copy(rank_hbm.at[pl.ds(base, TPW)], rk_v)
    pltpu.sync_copy(x_hbm.at[pl.ds(base, TPW)], rows_v)
    pltpu.async_copy(rows_v, xg_hbm.at[rk_v], sem).wait()


HC = 1024
NH = H // HC


def _moe_body(be_ref, act_ref, nlast_ref, xg_ref, w1_ref, w3_ref, w2_ref, out_ref):
    @pl.when(act_ref[pl.program_id(0)] == 1)
    def _():
        xb = xg_ref[...]
        h = jnp.dot(xb, w1_ref[0], preferred_element_type=jnp.float32)
        g = jnp.dot(xb, w3_ref[0], preferred_element_type=jnp.float32)
        act = (h * jax.nn.sigmoid(h)) * g
        part = jnp.dot(act, w2_ref[0], preferred_element_type=jnp.float32)
        hc = pl.program_id(1)

        @pl.when(hc == 0)
        def _():
            out_ref[...] = part

        @pl.when(hc != 0)
        def _():
            out_ref[...] += part


_moe_gemm = pl.pallas_call(
    _moe_body,
    grid_spec=pltpu.PrefetchScalarGridSpec(
        num_scalar_prefetch=3,
        grid=(NB, NH),
        in_specs=[
            pl.BlockSpec((BT, D), lambda b, h, be, act, nl: (nl[b], 0)),
            pl.BlockSpec((1, D, HC), lambda b, h, be, act, nl: (be[b], 0, h)),
            pl.BlockSpec((1, D, HC), lambda b, h, be, act, nl: (be[b], 0, h)),
            pl.BlockSpec((1, HC, D), lambda b, h, be, act, nl: (be[b], h, 0)),
        ],
        out_specs=pl.BlockSpec((BT, D), lambda b, h, be, act, nl: (nl[b], 0)),
    ),
    out_shape=jax.ShapeDtypeStruct((PT, D), jnp.float32),
    compiler_params=pltpu.CompilerParams(
        dimension_semantics=("arbitrary", "arbitrary"),
    ),
)


def _combine_body(rank_hbm, yg_hbm, out_hbm, rk_v, rows_v, sem):
    wid = lax.axis_index("s") * NC + lax.axis_index("c")
    base = wid * TPW
    pltpu.sync_copy(rank_hbm.at[pl.ds(base, TPW)], rk_v)
    pltpu.async_copy(yg_hbm.at[rk_v], rows_v, sem).wait()
    pltpu.sync_copy(rows_v, out_hbm.at[pl.ds(base, TPW)])


@functools.cache
def _sc_kernels():
    mesh = plsc.VectorSubcoreMesh(
        core_axis_name="c", subcore_axis_name="s", num_cores=NC, num_subcores=NS
    )
    dispatch = pl.kernel(
        _dispatch_body,
        out_type=jax.ShapeDtypeStruct((PT, D), jnp.float32),
        mesh=mesh,
        scratch_types=[
            pltpu.VMEM((TPW,), jnp.int32),
            pltpu.VMEM((TPW, D), jnp.float32),
            pltpu.SemaphoreType.DMA,
        ],
    )
    combine = pl.kernel(
        _combine_body,
        out_type=jax.ShapeDtypeStruct((T, D), jnp.float32),
        mesh=mesh,
        scratch_types=[
            pltpu.VMEM((TPW,), jnp.int32),
            pltpu.VMEM((TPW, D), jnp.float32),
            pltpu.SemaphoreType.DMA,
        ],
    )
    return dispatch, combine


@jax.jit
def kernel(x, Wg, W1, W2, W3):
    dispatch, combine = _sc_kernels()
    Bx, Sx, Dx = x.shape
    x_flat = x.reshape(T, D)
    rank2d, be_pad = _routing(x_flat, Wg)
    rank = rank2d.reshape(T)
    be = be_pad[0, :NB]
    act = be_pad[1, :NB]
    nlast = be_pad[2, :NB]
    xg = dispatch(rank, x_flat)
    yg = _moe_gemm(be, act, nlast, xg, W1, W3, W2)
    out = combine(rank, yg)
    return out.reshape(Bx, Sx, Dx)


# speedup vs baseline: 1.3139x; 1.3139x over previous
import functools

import jax
import jax.numpy as jnp
from jax import lax
from jax.experimental import pallas as pl
from jax.experimental.pallas import tpu as pltpu
from jax.experimental.pallas import tpu_sc as plsc

T = 2048
D = 768
E = 16
H = 2048
BT = 256
NB = T // BT + E
PT = NB * BT
NC = 2
NS = 16
NW = NC * NS
SLOTS = PT // NW
TPW = T // NW


def _routing_body(x_ref, wg_ref, rank_ref, be_ref):
    x = x_ref[...]
    wg = wg_ref[...]
    logits = jnp.dot(x, wg, preferred_element_type=jnp.float32)
    mx = jnp.max(logits, axis=1, keepdims=True)
    iota_e = lax.broadcasted_iota(jnp.int32, (T, E), 1)
    eid = jnp.min(jnp.where(logits == mx, iota_e, E), axis=1, keepdims=True)
    onehot = (iota_e == eid).astype(jnp.float32)

    counts = jnp.sum(onehot, axis=0, keepdims=True)
    ci = counts.astype(jnp.int32)
    pc = (((ci + BT - 1) // BT) * BT).astype(jnp.float32)
    ir = lax.broadcasted_iota(jnp.int32, (E, E), 0)
    ic = lax.broadcasted_iota(jnp.int32, (E, E), 1)
    excl = (ir < ic).astype(jnp.float32)
    pstarts = jnp.dot(pc, excl, preferred_element_type=jnp.float32)
    pends = pstarts + pc

    sel_start = jnp.sum(onehot * pstarts, axis=1)

    ir2 = lax.broadcasted_iota(jnp.int32, (BT, BT), 0)
    ic2 = lax.broadcasted_iota(jnp.int32, (BT, BT), 1)
    ltri = (ic2 < ir2).astype(jnp.float32)
    base = jnp.zeros((1, E), jnp.float32)
    for k in range(T // BT):
        blk = onehot[k * BT:(k + 1) * BT, :]
        cum = jnp.dot(ltri, blk, preferred_element_type=jnp.float32) + base
        r = jnp.sum(cum * blk, axis=1) + sel_start[k * BT:(k + 1) * BT]
        rank_ref[k, :] = r.astype(jnp.int32)
        base = base + jnp.sum(blk, axis=0, keepdims=True)

    total = pends[0:1, E - 1:E]
    bb_raw = lax.broadcasted_iota(jnp.int32, (1, 128), 1).astype(jnp.float32) * BT
    bb = jnp.minimum(bb_raw, total - 1.0)
    acc = jnp.zeros((1, 128), jnp.int32)
    for e in range(E):
        acc = acc + (bb >= pends[0:1, e:e + 1]).astype(jnp.int32)
    be_ref[0:1, :] = jnp.clip(acc, 0, E - 1)
    be_ref[1:2, :] = (bb_raw < total).astype(jnp.int32)
    nb_act = total * (1.0 / BT)
    biota = lax.broadcasted_iota(jnp.int32, (1, 128), 1).astype(jnp.float32)
    be_ref[2:3, :] = jnp.minimum(biota, nb_act - 1.0).astype(jnp.int32)


_routing = pl.pallas_call(
    _routing_body,
    out_shape=(
        jax.ShapeDtypeStruct((T // BT, BT), jnp.int32),
        jax.ShapeDtypeStruct((3, 128), jnp.int32),
    ),
)


def _dispatch_body(rank_hbm, x_hbm, xg_hbm, rk_v, rows_v, sem):
    wid = lax.axis_index("s") * NC + lax.axis_index("c")
    base = wid * TPW
    pltpu.sync_copy(rank_hbm.at[pl.ds(base, TPW)], rk_v)
    pltpu.sync_copy(x_hbm.at[pl.ds(base, TPW)], rows_v)
    pltpu.async_copy(rows_v, xg_hbm.at[rk_v], sem).wait()


def _moe_body(be_ref, act_ref, nlast_ref, xg_ref, w1_ref, w3_ref, w2_ref, out_ref):
    @pl.when(act_ref[pl.program_id(0)] == 1)
    def _():
        xb = xg_ref[...]
        h = jnp.dot(xb, w1_ref[0], preferred_element_type=jnp.float32)
        g = jnp.dot(xb, w3_ref[0], preferred_element_type=jnp.float32)
        act = (h * jax.nn.sigmoid(h)) * g
        out_ref[...] = jnp.dot(act, w2_ref[0], preferred_element_type=jnp.float32)


_moe_gemm = pl.pallas_call(
    _moe_body,
    grid_spec=pltpu.PrefetchScalarGridSpec(
        num_scalar_prefetch=3,
        grid=(NB,),
        in_specs=[
            pl.BlockSpec((BT, D), lambda b, be, act, nl: (nl[b], 0)),
            pl.BlockSpec((1, D, H), lambda b, be, act, nl: (be[b], 0, 0)),
            pl.BlockSpec((1, D, H), lambda b, be, act, nl: (be[b], 0, 0)),
            pl.BlockSpec((1, H, D), lambda b, be, act, nl: (be[b], 0, 0)),
        ],
        out_specs=pl.BlockSpec((BT, D), lambda b, be, act, nl: (nl[b], 0)),
    ),
    out_shape=jax.ShapeDtypeStruct((PT, D), jnp.float32),
    compiler_params=pltpu.CompilerParams(
        dimension_semantics=("arbitrary",),
    ),
)


def _combine_body(rank_hbm, yg_hbm, out_hbm, rk_v, rows_v, sem):
    wid = lax.axis_index("s") * NC + lax.axis_index("c")
    base = wid * TPW
    pltpu.sync_copy(rank_hbm.at[pl.ds(base, TPW)], rk_v)
    pltpu.async_copy(yg_hbm.at[rk_v], rows_v, sem).wait()
    pltpu.sync_copy(rows_v, out_hbm.at[pl.ds(base, TPW)])


@functools.cache
def _sc_kernels():
    mesh = plsc.VectorSubcoreMesh(
        core_axis_name="c", subcore_axis_name="s", num_cores=NC, num_subcores=NS
    )
    dispatch = pl.kernel(
        _dispatch_body,
        out_type=jax.ShapeDtypeStruct((PT, D), jnp.float32),
        mesh=mesh,
        scratch_types=[
            pltpu.VMEM((TPW,), jnp.int32),
            pltpu.VMEM((TPW, D), jnp.float32),
            pltpu.SemaphoreType.DMA,
        ],
    )
    combine = pl.kernel(
        _combine_body,
        out_type=jax.ShapeDtypeStruct((T, D), jnp.float32),
        mesh=mesh,
        scratch_types=[
            pltpu.VMEM((TPW,), jnp.int32),
            pltpu.VMEM((TPW, D), jnp.float32),
            pltpu.SemaphoreType.DMA,
        ],
    )
    return dispatch, combine


@jax.jit
def kernel(x, Wg, W1, W2, W3):
    dispatch, combine = _sc_kernels()
    Bx, Sx, Dx = x.shape
    x_flat = x.reshape(T, D)
    rank2d, be_pad = _routing(x_flat, Wg)
    rank = rank2d.reshape(T)
    be = be_pad[0, :NB]
    act = be_pad[1, :NB]
    nlast = be_pad[2, :NB]
    xg = dispatch(rank, x_flat)
    yg = _moe_gemm(be, act, nlast, xg, W1, W3, W2)
    out = combine(rank, yg)
    return out.reshape(Bx, Sx, Dx)


# speedup vs baseline: 1.3331x; 1.0146x over previous
import functools

import jax
import jax.numpy as jnp
from jax import lax
from jax.experimental import pallas as pl
from jax.experimental.pallas import tpu as pltpu
from jax.experimental.pallas import tpu_sc as plsc

T = 2048
D = 768
E = 16
H = 2048
BT = 256
NB = T // BT + E
PT = NB * BT
NC = 2
NS = 16
NW = NC * NS
SLOTS = PT // NW
TPW = T // NW


def _routing_body(x_ref, wg_ref, rank_ref, be_ref):
    x = x_ref[...]
    wg = wg_ref[...]
    logits = jnp.dot(x, wg, preferred_element_type=jnp.float32)
    mx = jnp.max(logits, axis=1, keepdims=True)
    iota_e = lax.broadcasted_iota(jnp.int32, (T, E), 1)
    eid = jnp.min(jnp.where(logits == mx, iota_e, E), axis=1, keepdims=True)
    onehot = (iota_e == eid).astype(jnp.float32)

    counts = jnp.sum(onehot, axis=0, keepdims=True)
    ci = counts.astype(jnp.int32)
    pc = (((ci + BT - 1) // BT) * BT).astype(jnp.float32)
    ir = lax.broadcasted_iota(jnp.int32, (E, E), 0)
    ic = lax.broadcasted_iota(jnp.int32, (E, E), 1)
    excl = (ir < ic).astype(jnp.float32)
    pstarts = jnp.dot(pc, excl, preferred_element_type=jnp.float32)
    pends = pstarts + pc

    sel_start = jnp.sum(onehot * pstarts, axis=1)

    RB = 128
    ir2 = lax.broadcasted_iota(jnp.int32, (RB, RB), 0)
    ic2 = lax.broadcasted_iota(jnp.int32, (RB, RB), 1)
    ltri = (ic2 < ir2).astype(jnp.float32)
    per_row = BT // RB
    base = jnp.zeros((1, E), jnp.float32)
    for k in range(T // RB):
        blk = onehot[k * RB:(k + 1) * RB, :]
        cum = jnp.dot(ltri, blk, preferred_element_type=jnp.float32) + base
        r = jnp.sum(cum * blk, axis=1) + sel_start[k * RB:(k + 1) * RB]
        col = (k % per_row) * RB
        rank_ref[k // per_row, col:col + RB] = r.astype(jnp.int32)
        base = base + jnp.sum(blk, axis=0, keepdims=True)

    total = pends[0:1, E - 1:E]
    bb_raw = lax.broadcasted_iota(jnp.int32, (1, 128), 1).astype(jnp.float32) * BT
    bb = jnp.minimum(bb_raw, total - 1.0)
    acc = jnp.zeros((1, 128), jnp.int32)
    for e in range(E):
        acc = acc + (bb >= pends[0:1, e:e + 1]).astype(jnp.int32)
    be_ref[0:1, :] = jnp.clip(acc, 0, E - 1)
    be_ref[1:2, :] = (bb_raw < total).astype(jnp.int32)
    nb_act = total * (1.0 / BT)
    biota = lax.broadcasted_iota(jnp.int32, (1, 128), 1).astype(jnp.float32)
    be_ref[2:3, :] = jnp.minimum(biota, nb_act - 1.0).astype(jnp.int32)


_routing = pl.pallas_call(
    _routing_body,
    out_shape=(
        jax.ShapeDtypeStruct((T // BT, BT), jnp.int32),
        jax.ShapeDtypeStruct((3, 128), jnp.int32),
    ),
)


HPW = TPW // 2


def _dispatch_body(rank_hbm, x_hbm, xg_hbm, rk0, rk1, rows0, rows1, sem0, sem1):
    wid = lax.axis_index("s") * NC + lax.axis_index("c")
    base = wid * TPW
    pltpu.sync_copy(rank_hbm.at[pl.ds(base, HPW)], rk0)
    pltpu.sync_copy(rank_hbm.at[pl.ds(base + HPW, HPW)], rk1)
    pltpu.sync_copy(x_hbm.at[pl.ds(base, HPW)], rows0)
    d0 = pltpu.async_copy(rows0, xg_hbm.at[rk0], sem0)
    pltpu.sync_copy(x_hbm.at[pl.ds(base + HPW, HPW)], rows1)
    d1 = pltpu.async_copy(rows1, xg_hbm.at[rk1], sem1)
    d0.wait()
    d1.wait()


def _moe_body(be_ref, act_ref, nlast_ref, xg_ref, w1_ref, w3_ref, w2_ref, out_ref):
    @pl.when(act_ref[pl.program_id(0)] == 1)
    def _():
        xb = xg_ref[...]
        h = jnp.dot(xb, w1_ref[0], preferred_element_type=jnp.float32)
        g = jnp.dot(xb, w3_ref[0], preferred_element_type=jnp.float32)
        act = (h * jax.nn.sigmoid(h)) * g
        out_ref[...] = jnp.dot(act, w2_ref[0], preferred_element_type=jnp.float32)


_moe_gemm = pl.pallas_call(
    _moe_body,
    grid_spec=pltpu.PrefetchScalarGridSpec(
        num_scalar_prefetch=3,
        grid=(NB,),
        in_specs=[
            pl.BlockSpec((BT, D), lambda b, be, act, nl: (nl[b], 0)),
            pl.BlockSpec((1, D, H), lambda b, be, act, nl: (be[b], 0, 0)),
            pl.BlockSpec((1, D, H), lambda b, be, act, nl: (be[b], 0, 0)),
            pl.BlockSpec((1, H, D), lambda b, be, act, nl: (be[b], 0, 0)),
        ],
        out_specs=pl.BlockSpec((BT, D), lambda b, be, act, nl: (nl[b], 0)),
    ),
    out_shape=jax.ShapeDtypeStruct((PT, D), jnp.float32),
    compiler_params=pltpu.CompilerParams(
        dimension_semantics=("arbitrary",),
    ),
)


def _combine_body(rank_hbm, yg_hbm, out_hbm, rk0, rk1, rows0, rows1, sem0, sem1):
    wid = lax.axis_index("s") * NC + lax.axis_index("c")
    base = wid * TPW
    pltpu.sync_copy(rank_hbm.at[pl.ds(base, HPW)], rk0)
    pltpu.sync_copy(rank_hbm.at[pl.ds(base + HPW, HPW)], rk1)
    g0 = pltpu.async_copy(yg_hbm.at[rk0], rows0, sem0)
    g1 = pltpu.async_copy(yg_hbm.at[rk1], rows1, sem1)
    g0.wait()
    pltpu.sync_copy(rows0, out_hbm.at[pl.ds(base, HPW)])
    g1.wait()
    pltpu.sync_copy(rows1, out_hbm.at[pl.ds(base + HPW, HPW)])


@functools.cache
def _sc_kernels():
    mesh = plsc.VectorSubcoreMesh(
        core_axis_name="c", subcore_axis_name="s", num_cores=NC, num_subcores=NS
    )
    sc_scratch = [
        pltpu.VMEM((HPW,), jnp.int32),
        pltpu.VMEM((HPW,), jnp.int32),
        pltpu.VMEM((HPW, D), jnp.float32),
        pltpu.VMEM((HPW, D), jnp.float32),
        pltpu.SemaphoreType.DMA,
        pltpu.SemaphoreType.DMA,
    ]
    dispatch = pl.kernel(
        _dispatch_body,
        out_type=jax.ShapeDtypeStruct((PT, D), jnp.float32),
        mesh=mesh,
        scratch_types=list(sc_scratch),
    )
    combine = pl.kernel(
        _combine_body,
        out_type=jax.ShapeDtypeStruct((T, D), jnp.float32),
        mesh=mesh,
        scratch_types=list(sc_scratch),
    )
    return dispatch, combine


@jax.jit
def kernel(x, Wg, W1, W2, W3):
    dispatch, combine = _sc_kernels()
    Bx, Sx, Dx = x.shape
    x_flat = x.reshape(T, D)
    rank2d, be_pad = _routing(x_flat, Wg)
    rank = rank2d.reshape(T)
    be = be_pad[0, :NB]
    act = be_pad[1, :NB]
    nlast = be_pad[2, :NB]
    xg = dispatch(rank, x_flat)
    yg = _moe_gemm(be, act, nlast, xg, W1, W3, W2)
    out = combine(rank, yg)
    return out.reshape(Bx, Sx, Dx)
